# baseline (speedup 1.0000x reference)
import jax
import jax.numpy as jnp
from jax.experimental import pallas as pl

NUM_HEAD = 8
R0, R1 = 2, 2
WX = 2 * R0 + 2
WY = 2 * R1 + 2
A = WX * WY


def _proj_kernel(x_ref, w_ref, o_ref):
    o_ref[...] = jnp.dot(x_ref[...], w_ref[...],
                         preferred_element_type=jnp.float32)


def _proj(x2d, wT):
    M, K = x2d.shape
    N = wT.shape[1]
    return pl.pallas_call(
        _proj_kernel,
        out_shape=jax.ShapeDtypeStruct((M, N), jnp.float32),
        grid=(M // 1152,),
        in_specs=[
            pl.BlockSpec((1152, K), lambda i: (i, 0)),
            pl.BlockSpec((K, N), lambda i: (0, 0)),
        ],
        out_specs=pl.BlockSpec((1152, N), lambda i: (i, 0)),
    )(x2d, wT)


def kernel(x, max_offset, Wq, Wk, Wv, Wproj):
    Bb, Hh, Ww, C = x.shape
    HEAD_DIM = C // NUM_HEAD
    SCALE = HEAD_DIM ** -0.5
    N = Hh * Ww
    xf = x.reshape(Bb, N, C)
    q = (xf @ Wq.T).reshape(Bb, N, NUM_HEAD, HEAD_DIM)
    k = (xf @ Wk.T).reshape(Bb, N, NUM_HEAD, HEAD_DIM)
    v = (xf @ Wv.T).reshape(Bb, N, NUM_HEAD, HEAD_DIM)
    mo = max_offset.reshape(Bb, N, 2)
    ox = jnp.clip(mo[..., 0], R0, Ww - 1 - R0 - 0.001)
    oy = jnp.clip(mo[..., 1], R1, Hh - 1 - R1 - 0.001)
    mxf = jnp.floor(ox)
    myf = jnp.floor(oy)
    mx = mxf.astype(jnp.int32)
    my = myf.astype(jnp.int32)
    fx = ox - mxf
    fy = oy - myf
    dxs = jnp.arange(-R0, R0 + 2)
    dys = jnp.arange(-R1, R1 + 2)
    pos_x = mx[..., None, None] + dxs[None, :]
    pos_y = my[..., None, None] + dys[:, None]
    idx = (pos_y * Ww + pos_x).reshape(Bb, N, A)
    kg = jax.vmap(lambda t, i: t[i])(k, idx)
    vg = jax.vmap(lambda t, i: t[i])(v, idx)
    attn = jnp.einsum('bnhd,bnahd->bnha', q, kg) * SCALE
    attn = attn.reshape(Bb, N, NUM_HEAD, WY, WX)
    out = jnp.zeros_like(attn)
    wbl = {(0, 0): (1 - fy) * (1 - fx), (0, 1): (1 - fy) * fx,
           (1, 0): fy * (1 - fx), (1, 1): fy * fx}
    for sy in (0, 1):
        for sx in (0, 1):
            sub = attn[..., sy:sy + 2 * R1 + 1, sx:sx + 2 * R0 + 1]
            sm = jax.nn.softmax(sub.reshape(Bb, N, NUM_HEAD, -1),
                                axis=-1).reshape(sub.shape)
            w = wbl[(sy, sx)][:, :, None, None, None]
            out = out.at[..., sy:sy + 2 * R1 + 1, sx:sx + 2 * R0 + 1].add(w * sm)
    attn_flat = out.reshape(Bb, N, NUM_HEAD, A)
    agg = jnp.einsum('bnha,bnahd->bnhd', attn_flat, vg).reshape(
        Bb, N, NUM_HEAD * HEAD_DIM)
    y = jax.vmap(lambda a: _proj(a, Wproj.T))(agg)
    return y.reshape(Bb, Hh, Ww, -1)


# speedup vs baseline: 11.7445x; 11.7445x over previous
import functools

import jax
import jax.numpy as jnp
from jax.experimental import pallas as pl
from jax.experimental.pallas import tpu as pltpu

NUM_HEAD = 8
R0, R1 = 2, 2
WX = 2 * R0 + 2
WY = 2 * R1 + 2
TQ = 16
_INTERPRET = False


def _qkv_kernel(x_ref, w_ref, o_ref):
    o_ref[...] = jnp.dot(x_ref[...], w_ref[...],
                         preferred_element_type=jnp.float32)


def _attn_kernel(pb_ref, q_ref, k_ref, v_ref, fx_ref, fy_ref, wp_ref,
                 o_ref, kg_s, vg_s, *, Ww, C):
    HEAD_DIM = C // NUM_HEAD
    SCALE = HEAD_DIM ** -0.5
    SLOT = 8 * WY

    b = pl.program_id(0)
    i = pl.program_id(1)

    @pl.when(jnp.logical_and(b == 0, i == 0))
    def _init():
        kg_s[...] = jnp.zeros_like(kg_s)
        vg_s[...] = jnp.zeros_like(vg_s)

    def gather_one(iq, _):
        p = pb_ref[0, 0, iq]
        ph = (p // 8) * 8
        pr = p - ph
        shift = 16 - pr
        for dy in range(WY):
            src = pl.ds(ph + dy * Ww, 16)
            dst = pl.ds(iq * SLOT + dy * 8, WX)
            ka = pltpu.roll(k_ref[0, src, :], shift, 0)
            va = pltpu.roll(v_ref[0, src, :], shift, 0)
            kg_s[dst, :] = ka[:WX, :]
            vg_s[dst, :] = va[:WX, :]
        return 0

    jax.lax.fori_loop(0, TQ, gather_one, 0, unroll=2)

    q_blk = q_ref[0]
    qrep = jnp.broadcast_to(q_blk[:, None, :], (TQ, SLOT, C)).reshape(
        TQ * SLOT, C)
    prod = kg_s[...] * qrep
    dlane = jax.lax.broadcasted_iota(jnp.int32, (C, NUM_HEAD), 0)
    hlane = jax.lax.broadcasted_iota(jnp.int32, (C, NUM_HEAD), 1)
    hm = (dlane // HEAD_DIM == hlane).astype(jnp.float32)
    scores = jnp.dot(prod, hm, preferred_element_type=jnp.float32) * SCALE

    st = jnp.swapaxes(scores.reshape(TQ, SLOT, NUM_HEAD), 1, 2)
    X = st.reshape(TQ * NUM_HEAD, SLOT)

    lane = jax.lax.broadcasted_iota(jnp.int32, (1, SLOT), 1)
    ldy = lane // 8
    ldx = lane % 8
    fx = jnp.broadcast_to(fx_ref[0][:, None, :],
                          (TQ, NUM_HEAD, 1)).reshape(TQ * NUM_HEAD, 1)
    fy = jnp.broadcast_to(fy_ref[0][:, None, :],
                          (TQ, NUM_HEAD, 1)).reshape(TQ * NUM_HEAD, 1)
    W_acc = jnp.zeros_like(X)
    for sy in (0, 1):
        for sx in (0, 1):
            m = ((ldy >= sy) & (ldy <= sy + 2 * R1) &
                 (ldx >= sx) & (ldx <= sx + 2 * R0))
            Xm = jnp.where(m, X, -1e30)
            mx = jnp.max(Xm, axis=-1, keepdims=True)
            e = jnp.exp(Xm - mx)
            s = jnp.sum(e, axis=-1, keepdims=True)
            wy = fy if sy else (1.0 - fy)
            wx = fx if sx else (1.0 - fx)
            W_acc = W_acc + (wy * wx / s) * e

    Wb = jnp.swapaxes(W_acc.reshape(TQ, NUM_HEAD, SLOT), 1, 2).reshape(
        TQ * SLOT, NUM_HEAD)
    broad = jnp.dot(Wb, hm.T, preferred_element_type=jnp.float32)
    weighted = broad * vg_s[...]
    agg = jnp.sum(weighted.reshape(TQ, SLOT, C), axis=1)
    o_ref[0] = jnp.dot(agg, wp_ref[...], preferred_element_type=jnp.float32)


def kernel(x, max_offset, Wq, Wk, Wv, Wproj):
    Bb, Hh, Ww, C = x.shape
    N = Hh * Ww
    NB = N // TQ

    x2d = x.reshape(Bb * N, C)
    wcat = jnp.concatenate([Wq.T, Wk.T, Wv.T], axis=1)
    MB = next(m for m in (1152, 512, 256, 128, 64, 32, 16, 8)
              if (Bb * N) % m == 0)
    qkv = pl.pallas_call(
        _qkv_kernel,
        out_shape=jax.ShapeDtypeStruct((Bb * N, 3 * C), jnp.float32),
        grid=(Bb * N // MB,),
        in_specs=[
            pl.BlockSpec((MB, C), lambda m: (m, 0)),
            pl.BlockSpec((C, 3 * C), lambda m: (0, 0)),
        ],
        out_specs=pl.BlockSpec((MB, 3 * C), lambda m: (m, 0)),
        interpret=_INTERPRET,
    )(x2d, wcat)
    q = qkv[:, :C].reshape(Bb, N, C)
    k = qkv[:, C:2 * C].reshape(Bb, N, C)
    v = qkv[:, 2 * C:].reshape(Bb, N, C)

    mo = max_offset.reshape(Bb, N, 2)
    ox = jnp.clip(mo[..., 0], R0, Ww - 1 - R0 - 0.001)
    oy = jnp.clip(mo[..., 1], R1, Hh - 1 - R1 - 0.001)
    mxf = jnp.floor(ox)
    myf = jnp.floor(oy)
    fx = (ox - mxf).reshape(Bb, N, 1)
    fy = (oy - myf).reshape(Bb, N, 1)
    pbase = ((myf.astype(jnp.int32) - R1) * Ww +
             (mxf.astype(jnp.int32) - R0)).reshape(Bb * NB, 1, TQ)

    kp = jnp.concatenate([k, jnp.zeros((Bb, 16, C), jnp.float32)], axis=1)
    vp = jnp.concatenate([v, jnp.zeros((Bb, 16, C), jnp.float32)], axis=1)
    SLOT = 8 * WY
    y2 = pl.pallas_call(
        functools.partial(_attn_kernel, Ww=Ww, C=C),
        out_shape=jax.ShapeDtypeStruct((Bb, N, C), jnp.float32),
        grid=(Bb, NB),
        in_specs=[
            pl.BlockSpec((1, 1, TQ), lambda b, i, NB=NB: (b * NB + i, 0, 0),
                         memory_space=pltpu.SMEM),
            pl.BlockSpec((1, TQ, C), lambda b, i: (b, i, 0)),
            pl.BlockSpec((1, N + 16, C), lambda b, i: (b, 0, 0)),
            pl.BlockSpec((1, N + 16, C), lambda b, i: (b, 0, 0)),
            pl.BlockSpec((1, TQ, 1), lambda b, i: (b, i, 0)),
            pl.BlockSpec((1, TQ, 1), lambda b, i: (b, i, 0)),
            pl.BlockSpec((C, C), lambda b, i: (0, 0)),
        ],
        out_specs=pl.BlockSpec((1, TQ, C), lambda b, i: (b, i, 0)),
        scratch_shapes=[
            pltpu.VMEM((TQ * SLOT, C), jnp.float32),
            pltpu.VMEM((TQ * SLOT, C), jnp.float32),
        ],
        interpret=_INTERPRET,
    )(pbase, q, kp, vp, fx, fy, Wproj.T)
    return y2.reshape(Bb, Hh, Ww, C)


# speedup vs baseline: 14.7920x; 1.2595x over previous
import functools

import jax
import jax.numpy as jnp
from jax.experimental import pallas as pl
from jax.experimental.pallas import tpu as pltpu

NUM_HEAD = 8
R0, R1 = 2, 2
WX = 2 * R0 + 2
WY = 2 * R1 + 2
TQ = 32
_INTERPRET = False


def _qkv_kernel(x_ref, w_ref, o_ref):
    o_ref[...] = jnp.dot(x_ref[...], w_ref[...],
                         preferred_element_type=jnp.float32)


def _attn_kernel(pb_ref, q_ref, kvh_ref, fx_ref, fy_ref, wp_ref,
                 o_ref, kvg_s, kv_vm, sem, *, Ww, C):
    HEAD_DIM = C // NUM_HEAD
    SCALE = HEAD_DIM ** -0.5
    SLOT = 8 * WY

    b = pl.program_id(0)
    i = pl.program_id(1)

    @pl.when(jnp.logical_and(b == 0, i == 0))
    def _init():
        kvg_s[...] = jnp.zeros_like(kvg_s)

    @pl.when(i == 0)
    def _stage_kv():
        cp = pltpu.make_async_copy(kvh_ref.at[b], kv_vm, sem)
        cp.start()
        cp.wait()

    def gather_one(iq, _):
        p = pb_ref[0, 0, iq]
        ph = (p // 8) * 8
        shift = 16 - (p - ph)
        for dy in range(WY):
            src = pl.ds(ph + dy * Ww, 16)
            dst = pl.ds(iq * SLOT + dy * 8, WX)
            a = pltpu.roll(kv_vm[src, :], shift, 0)
            kvg_s[dst, :] = a[:WX, :]
        return 0

    jax.lax.fori_loop(0, TQ, gather_one, 0, unroll=4)

    q_blk = q_ref[0]
    qrep = jnp.broadcast_to(q_blk[:, None, :], (TQ, SLOT, C)).reshape(
        TQ * SLOT, C)
    prod = kvg_s[:, :C] * qrep
    dlane = jax.lax.broadcasted_iota(jnp.int32, (C, NUM_HEAD), 0)
    hlane = jax.lax.broadcasted_iota(jnp.int32, (C, NUM_HEAD), 1)
    hm = (dlane // HEAD_DIM == hlane).astype(jnp.float32)
    scores = jnp.dot(prod, hm, preferred_element_type=jnp.float32) * SCALE

    st = jnp.swapaxes(scores.reshape(TQ, SLOT, NUM_HEAD), 1, 2)
    X = st.reshape(TQ * NUM_HEAD, SLOT)

    lane = jax.lax.broadcasted_iota(jnp.int32, (1, SLOT), 1)
    ldy = lane // 8
    ldx = lane % 8
    fx = jnp.broadcast_to(fx_ref[0][:, None, :],
                          (TQ, NUM_HEAD, 1)).reshape(TQ * NUM_HEAD, 1)
    fy = jnp.broadcast_to(fy_ref[0][:, None, :],
                          (TQ, NUM_HEAD, 1)).reshape(TQ * NUM_HEAD, 1)
    W_acc = jnp.zeros_like(X)
    for sy in (0, 1):
        for sx in (0, 1):
            m = ((ldy >= sy) & (ldy <= sy + 2 * R1) &
                 (ldx >= sx) & (ldx <= sx + 2 * R0))
            Xm = jnp.where(m, X, -1e30)
            mx = jnp.max(Xm, axis=-1, keepdims=True)
            e = jnp.exp(Xm - mx)
            s = jnp.sum(e, axis=-1, keepdims=True)
            wy = fy if sy else (1.0 - fy)
            wx = fx if sx else (1.0 - fx)
            W_acc = W_acc + (wy * wx / s) * e

    Wb = jnp.swapaxes(W_acc.reshape(TQ, NUM_HEAD, SLOT), 1, 2).reshape(
        TQ * SLOT, NUM_HEAD)
    broad = jnp.dot(Wb, hm.T, preferred_element_type=jnp.float32)
    weighted = broad * kvg_s[:, C:]
    agg = jnp.sum(weighted.reshape(TQ, SLOT, C), axis=1)
    o_ref[0] = jnp.dot(agg, wp_ref[...], preferred_element_type=jnp.float32)


def kernel(x, max_offset, Wq, Wk, Wv, Wproj):
    Bb, Hh, Ww, C = x.shape
    N = Hh * Ww
    NB = N // TQ

    x2d = x.reshape(Bb * N, C)
    wcat = jnp.concatenate([Wq.T, Wk.T, Wv.T], axis=1)
    MB = next(m for m in (1152, 512, 256, 128, 64, 32, 16, 8)
              if (Bb * N) % m == 0)
    qkv = pl.pallas_call(
        _qkv_kernel,
        out_shape=jax.ShapeDtypeStruct((Bb * N, 3 * C), jnp.float32),
        grid=(Bb * N // MB,),
        in_specs=[
            pl.BlockSpec((MB, C), lambda m: (m, 0)),
            pl.BlockSpec((C, 3 * C), lambda m: (0, 0)),
        ],
        out_specs=pl.BlockSpec((MB, 3 * C), lambda m: (m, 0)),
        interpret=_INTERPRET,
    )(x2d, wcat)
    q = qkv[:, :C].reshape(Bb, N, C)
    kv = qkv[:, C:].reshape(Bb, N, 2 * C)

    mo = max_offset.reshape(Bb, N, 2)
    ox = jnp.clip(mo[..., 0], R0, Ww - 1 - R0 - 0.001)
    oy = jnp.clip(mo[..., 1], R1, Hh - 1 - R1 - 0.001)
    mxf = jnp.floor(ox)
    myf = jnp.floor(oy)
    fx = (ox - mxf).reshape(Bb, N, 1)
    fy = (oy - myf).reshape(Bb, N, 1)
    pbase = ((myf.astype(jnp.int32) - R1) * Ww +
             (mxf.astype(jnp.int32) - R0)).reshape(Bb * NB, 1, TQ)

    kvp = jnp.concatenate([kv, jnp.zeros((Bb, 16, 2 * C), jnp.float32)],
                          axis=1)
    SLOT = 8 * WY
    y2 = pl.pallas_call(
        functools.partial(_attn_kernel, Ww=Ww, C=C),
        out_shape=jax.ShapeDtypeStruct((Bb, N, C), jnp.float32),
        grid=(Bb, NB),
        in_specs=[
            pl.BlockSpec((1, 1, TQ), lambda b, i, NB=NB: (b * NB + i, 0, 0),
                         memory_space=pltpu.SMEM),
            pl.BlockSpec((1, TQ, C), lambda b, i: (b, i, 0)),
            pl.BlockSpec(memory_space=pltpu.MemorySpace.HBM),
            pl.BlockSpec((1, TQ, 1), lambda b, i: (b, i, 0)),
            pl.BlockSpec((1, TQ, 1), lambda b, i: (b, i, 0)),
            pl.BlockSpec((C, C), lambda b, i: (0, 0)),
        ],
        out_specs=pl.BlockSpec((1, TQ, C), lambda b, i: (b, i, 0)),
        scratch_shapes=[
            pltpu.VMEM((TQ * SLOT, 2 * C), jnp.float32),
            pltpu.VMEM((N + 16, 2 * C), jnp.float32),
            pltpu.SemaphoreType.DMA,
        ],
        interpret=_INTERPRET,
    )(pbase, q, kvp, fx, fy, Wproj.T)
    return y2.reshape(Bb, Hh, Ww, C)


# speedup vs baseline: 15.1663x; 1.0253x over previous
import functools

import jax
import jax.numpy as jnp
from jax.experimental import pallas as pl
from jax.experimental.pallas import tpu as pltpu

NUM_HEAD = 8
R0, R1 = 2, 2
WX = 2 * R0 + 2
WY = 2 * R1 + 2
TQ = 32
SLOT = 8 * WY
_INTERPRET = False


def _mm_kernel(x_ref, w_ref, o_ref):
    o_ref[...] = jnp.dot(x_ref[...], w_ref[...],
                         preferred_element_type=jnp.float32)


def _mm(x2d, w, interpret):
    M, K = x2d.shape
    Nn = w.shape[1]
    MB = next(m for m in (1152, 512, 256, 128, 64, 32, 16, 8) if M % m == 0)
    return pl.pallas_call(
        _mm_kernel,
        out_shape=jax.ShapeDtypeStruct((M, Nn), jnp.float32),
        grid=(M // MB,),
        in_specs=[
            pl.BlockSpec((MB, K), lambda m: (m, 0)),
            pl.BlockSpec((K, Nn), lambda m: (0, 0)),
        ],
        out_specs=pl.BlockSpec((MB, Nn), lambda m: (m, 0)),
        interpret=interpret,
    )(x2d, w)


def _attn_kernel(pb_ref, q_ref, kvh_ref, fx_ref, fy_ref, hm_ref, bias_ref,
                 o_ref, kvg_s, kv_vm, sem, *, Ww, C):
    b = pl.program_id(0)
    i = pl.program_id(1)

    @pl.when(jnp.logical_and(b == 0, i == 0))
    def _init():
        kvg_s[...] = jnp.zeros_like(kvg_s)

    @pl.when(i == 0)
    def _stage_kv():
        cp = pltpu.make_async_copy(kvh_ref.at[b], kv_vm, sem)
        cp.start()
        cp.wait()

    def gather_one(iq, _):
        p = pb_ref[0, 0, iq]
        ph = (p // 8) * 8
        shift = 16 - (p - ph)
        for dy in range(WY):
            src = pl.ds(ph + dy * Ww, 16)
            dst = pl.ds(iq * SLOT + dy * 8, WX)
            a = pltpu.roll(kv_vm[src, :], shift, 0)
            kvg_s[dst, :] = a[:WX, :]
        return 0

    jax.lax.fori_loop(0, TQ, gather_one, 0, unroll=4)

    prod3 = (kvg_s[:, :C].reshape(TQ, SLOT, C) * q_ref[0][:, None, :])
    scores = jnp.dot(prod3.reshape(TQ * SLOT, C), hm_ref[...],
                     preferred_element_type=jnp.float32)

    st = jnp.swapaxes(scores.reshape(TQ, SLOT, NUM_HEAD), 1, 2)
    X = st.reshape(TQ * NUM_HEAD, SLOT)

    fx = jnp.broadcast_to(fx_ref[0][:, None, :],
                          (TQ, NUM_HEAD, 1)).reshape(TQ * NUM_HEAD, 1)
    fy = jnp.broadcast_to(fy_ref[0][:, None, :],
                          (TQ, NUM_HEAD, 1)).reshape(TQ * NUM_HEAD, 1)
    W_acc = jnp.zeros_like(X)
    for s_idx, (sy, sx) in enumerate(((0, 0), (0, 1), (1, 0), (1, 1))):
        e = jnp.exp(X + bias_ref[s_idx:s_idx + 1, :])
        ssum = jnp.sum(e, axis=-1, keepdims=True)
        wy = fy if sy else (1.0 - fy)
        wx = fx if sx else (1.0 - fx)
        W_acc = W_acc + (wy * wx / ssum) * e

    Wb = jnp.swapaxes(W_acc.reshape(TQ, NUM_HEAD, SLOT), 1, 2).reshape(
        TQ * SLOT, NUM_HEAD)
    broad = jnp.dot(Wb, hm_ref[...].T, preferred_element_type=jnp.float32)
    weighted = broad * kvg_s[:, C:]
    o_ref[0] = jnp.sum(weighted.reshape(TQ, SLOT, C), axis=1)


def kernel(x, max_offset, Wq, Wk, Wv, Wproj):
    Bb, Hh, Ww, C = x.shape
    HEAD_DIM = C // NUM_HEAD
    SCALE = HEAD_DIM ** -0.5
    N = Hh * Ww
    NB = N // TQ

    x2d = x.reshape(Bb * N, C)
    wcat = jnp.concatenate([Wq.T * SCALE, Wk.T, Wv.T], axis=1)
    qkv = _mm(x2d, wcat, _INTERPRET)
    q = qkv[:, :C].reshape(Bb, N, C)
    kv = qkv[:, C:].reshape(Bb, N, 2 * C)

    mo = max_offset.reshape(Bb, N, 2)
    ox = jnp.clip(mo[..., 0], R0, Ww - 1 - R0 - 0.001)
    oy = jnp.clip(mo[..., 1], R1, Hh - 1 - R1 - 0.001)
    mxf = jnp.floor(ox)
    myf = jnp.floor(oy)
    fx = (ox - mxf).reshape(Bb, N, 1)
    fy = (oy - myf).reshape(Bb, N, 1)
    pbase = ((myf.astype(jnp.int32) - R1) * Ww +
             (mxf.astype(jnp.int32) - R0)).reshape(Bb * NB, 1, TQ)

    dl = jnp.arange(C)[:, None]
    hm = (dl // HEAD_DIM == jnp.arange(NUM_HEAD)[None, :]).astype(jnp.float32)
    ldy = jnp.arange(SLOT)[None, :] // 8
    ldx = jnp.arange(SLOT)[None, :] % 8
    biases = []
    for sy, sx in ((0, 0), (0, 1), (1, 0), (1, 1)):
        m = ((ldy >= sy) & (ldy <= sy + 2 * R1) &
             (ldx >= sx) & (ldx <= sx + 2 * R0))
        biases.append(jnp.where(m, 0.0, -1e30).astype(jnp.float32))
    bias = jnp.concatenate(biases + biases, axis=0)

    kvp = jnp.concatenate([kv, jnp.zeros((Bb, 16, 2 * C), jnp.float32)],
                          axis=1)
    agg = pl.pallas_call(
        functools.partial(_attn_kernel, Ww=Ww, C=C),
        out_shape=jax.ShapeDtypeStruct((Bb, N, C), jnp.float32),
        grid=(Bb, NB),
        in_specs=[
            pl.BlockSpec((1, 1, TQ), lambda b, i, NB=NB: (b * NB + i, 0, 0),
                         memory_space=pltpu.SMEM),
            pl.BlockSpec((1, TQ, C), lambda b, i: (b, i, 0)),
            pl.BlockSpec(memory_space=pltpu.MemorySpace.HBM),
            pl.BlockSpec((1, TQ, 1), lambda b, i: (b, i, 0)),
            pl.BlockSpec((1, TQ, 1), lambda b, i: (b, i, 0)),
            pl.BlockSpec((C, NUM_HEAD), lambda b, i: (0, 0)),
            pl.BlockSpec((8, SLOT), lambda b, i: (0, 0)),
        ],
        out_specs=pl.BlockSpec((1, TQ, C), lambda b, i: (b, i, 0)),
        scratch_shapes=[
            pltpu.VMEM((TQ * SLOT, 2 * C), jnp.float32),
            pltpu.VMEM((N + 16, 2 * C), jnp.float32),
            pltpu.SemaphoreType.DMA,
        ],
        interpret=_INTERPRET,
    )(pbase, q, kvp, fx, fy, hm, bias)

    y = _mm(agg.reshape(Bb * N, C), Wproj.T, _INTERPRET)
    return y.reshape(Bb, Hh, Ww, C)


# speedup vs baseline: 15.5938x; 1.0282x over previous
import functools

import jax
import jax.numpy as jnp
from jax.experimental import pallas as pl
from jax.experimental.pallas import tpu as pltpu

NUM_HEAD = 8
R0, R1 = 2, 2
WX = 2 * R0 + 2
WY = 2 * R1 + 2
TQ = 64
SLOT = 8 * WY
_INTERPRET = False


def _mm_kernel(x_ref, w_ref, o_ref):
    o_ref[...] = jnp.dot(x_ref[...], w_ref[...],
                         preferred_element_type=jnp.float32)


def _mm(x2d, w, interpret):
    M, K = x2d.shape
    Nn = w.shape[1]
    MB = next(m for m in (1152, 512, 256, 128, 64, 32, 16, 8) if M % m == 0)
    return pl.pallas_call(
        _mm_kernel,
        out_shape=jax.ShapeDtypeStruct((M, Nn), jnp.float32),
        grid=(M // MB,),
        in_specs=[
            pl.BlockSpec((MB, K), lambda m: (m, 0)),
            pl.BlockSpec((K, Nn), lambda m: (0, 0)),
        ],
        out_specs=pl.BlockSpec((MB, Nn), lambda m: (m, 0)),
        interpret=interpret,
    )(x2d, w)


def _attn_kernel(pb_ref, q_ref, kvh_ref, fx_ref, fy_ref, hm_ref, bias_ref,
                 o_ref, kvg_s, kv_vm, sem, *, Ww, C):
    b = pl.program_id(0)
    i = pl.program_id(1)

    @pl.when(jnp.logical_and(b == 0, i == 0))
    def _init():
        kvg_s[...] = jnp.zeros_like(kvg_s)

    @pl.when(i == 0)
    def _stage_kv():
        cp = pltpu.make_async_copy(kvh_ref.at[b], kv_vm, sem)
        cp.start()
        cp.wait()

    def gather_one(iq, _):
        p = pb_ref[0, 0, iq]
        ph = (p // 8) * 8
        shift = 16 - (p - ph)
        for dy in range(WY):
            src = pl.ds(ph + dy * Ww, 16)
            dst = pl.ds(iq * SLOT + dy * 8, WX)
            a = pltpu.roll(kv_vm[src, :], shift, 0)
            kvg_s[dst, :] = a[:WX, :]
        return 0

    jax.lax.fori_loop(0, TQ, gather_one, 0, unroll=8)

    prod3 = (kvg_s[:, :C].reshape(TQ, SLOT, C) * q_ref[0][:, None, :])
    scores = jnp.dot(prod3.reshape(TQ * SLOT, C), hm_ref[...],
                     preferred_element_type=jnp.float32)

    st = jnp.swapaxes(scores.reshape(TQ, SLOT, NUM_HEAD), 1, 2)
    X = st.reshape(TQ * NUM_HEAD, SLOT)

    fx = jnp.broadcast_to(fx_ref[0][:, None, :],
                          (TQ, NUM_HEAD, 1)).reshape(TQ * NUM_HEAD, 1)
    fy = jnp.broadcast_to(fy_ref[0][:, None, :],
                          (TQ, NUM_HEAD, 1)).reshape(TQ * NUM_HEAD, 1)
    W_acc = jnp.zeros_like(X)
    for s_idx, (sy, sx) in enumerate(((0, 0), (0, 1), (1, 0), (1, 1))):
        e = jnp.exp(X + bias_ref[s_idx:s_idx + 1, :])
        ssum = jnp.sum(e, axis=-1, keepdims=True)
        wy = fy if sy else (1.0 - fy)
        wx = fx if sx else (1.0 - fx)
        W_acc = W_acc + (wy * wx / ssum) * e

    Wb = jnp.swapaxes(W_acc.reshape(TQ, NUM_HEAD, SLOT), 1, 2).reshape(
        TQ * SLOT, NUM_HEAD)
    broad = jnp.dot(Wb, hm_ref[...].T, preferred_element_type=jnp.float32)
    weighted = broad * kvg_s[:, C:]
    o_ref[0] = jnp.sum(weighted.reshape(TQ, SLOT, C), axis=1)


def kernel(x, max_offset, Wq, Wk, Wv, Wproj):
    Bb, Hh, Ww, C = x.shape
    HEAD_DIM = C // NUM_HEAD
    SCALE = HEAD_DIM ** -0.5
    N = Hh * Ww
    NB = N // TQ

    x2d = x.reshape(Bb * N, C)
    wcat = jnp.concatenate([Wq.T * SCALE, Wk.T, Wv.T], axis=1)
    qkv = _mm(x2d, wcat, _INTERPRET)
    q = qkv[:, :C].reshape(Bb, N, C)
    kv = qkv[:, C:].reshape(Bb, N, 2 * C)

    mo = max_offset.reshape(Bb, N, 2)
    ox = jnp.clip(mo[..., 0], R0, Ww - 1 - R0 - 0.001)
    oy = jnp.clip(mo[..., 1], R1, Hh - 1 - R1 - 0.001)
    mxf = jnp.floor(ox)
    myf = jnp.floor(oy)
    fx = (ox - mxf).reshape(Bb, N, 1)
    fy = (oy - myf).reshape(Bb, N, 1)
    pbase = ((myf.astype(jnp.int32) - R1) * Ww +
             (mxf.astype(jnp.int32) - R0)).reshape(Bb * NB, 1, TQ)

    dl = jnp.arange(C)[:, None]
    hm = (dl // HEAD_DIM == jnp.arange(NUM_HEAD)[None, :]).astype(jnp.float32)
    ldy = jnp.arange(SLOT)[None, :] // 8
    ldx = jnp.arange(SLOT)[None, :] % 8
    biases = []
    for sy, sx in ((0, 0), (0, 1), (1, 0), (1, 1)):
        m = ((ldy >= sy) & (ldy <= sy + 2 * R1) &
             (ldx >= sx) & (ldx <= sx + 2 * R0))
        biases.append(jnp.where(m, 0.0, -1e30).astype(jnp.float32))
    bias = jnp.concatenate(biases + biases, axis=0)

    kvp = jnp.concatenate([kv, jnp.zeros((Bb, 16, 2 * C), jnp.float32)],
                          axis=1)
    agg = pl.pallas_call(
        functools.partial(_attn_kernel, Ww=Ww, C=C),
        out_shape=jax.ShapeDtypeStruct((Bb, N, C), jnp.float32),
        grid=(Bb, NB),
        in_specs=[
            pl.BlockSpec((1, 1, TQ), lambda b, i, NB=NB: (b * NB + i, 0, 0),
                         memory_space=pltpu.SMEM),
            pl.BlockSpec((1, TQ, C), lambda b, i: (b, i, 0)),
            pl.BlockSpec(memory_space=pltpu.MemorySpace.HBM),
            pl.BlockSpec((1, TQ, 1), lambda b, i: (b, i, 0)),
            pl.BlockSpec((1, TQ, 1), lambda b, i: (b, i, 0)),
            pl.BlockSpec((C, NUM_HEAD), lambda b, i: (0, 0)),
            pl.BlockSpec((8, SLOT), lambda b, i: (0, 0)),
        ],
        out_specs=pl.BlockSpec((1, TQ, C), lambda b, i: (b, i, 0)),
        scratch_shapes=[
            pltpu.VMEM((TQ * SLOT, 2 * C), jnp.float32),
            pltpu.VMEM((N + 16, 2 * C), jnp.float32),
            pltpu.SemaphoreType.DMA,
        ],
        interpret=_INTERPRET,
    )(pbase, q, kvp, fx, fy, hm, bias)

    y = _mm(agg.reshape(Bb * N, C), Wproj.T, _INTERPRET)
    return y.reshape(Bb, Hh, Ww, C)
